# baseline (speedup 1.0000x reference)
import jax
import jax.numpy as jnp
from jax import lax
from jax.experimental import pallas as pl
from jax.experimental.pallas import tpu as pltpu

DIM = 512
CAP = 100000
BR = 2000
NBLK = CAP // BR


def _nearest_d8_2d(x):
    f = jnp.round(x)
    d = x - f
    ad = jnp.abs(d)
    m = jnp.max(ad, axis=-1, keepdims=True)
    col = lax.broadcasted_iota(jnp.int32, x.shape, 1)
    k = jnp.min(jnp.where(ad == m, col, jnp.int32(8)), axis=-1, keepdims=True)
    oh = (col == k).astype(x.dtype)
    d_k = jnp.sum(d * oh, axis=-1, keepdims=True)
    corr = jnp.where(d_k >= 0.0, 1.0, -1.0)
    f_fixed = f + oh * corr
    parity_odd = jnp.mod(jnp.sum(f, axis=-1, keepdims=True), 2.0) != 0.0
    return jnp.where(parity_odd, f_fixed, f)


def _anchor_body(q_ref, out_ref):
    x = q_ref[...] * 10.0
    y0 = _nearest_d8_2d(x)
    y1 = _nearest_d8_2d(x - 0.5) + 0.5
    d0 = jnp.sum((x - y0) ** 2, axis=-1, keepdims=True)
    d1 = jnp.sum((x - y1) ** 2, axis=-1, keepdims=True)
    out_ref[...] = jnp.where(d0 <= d1, y0, y1) * 0.1


def _retrieve_body(anchor_ref, keys_ref, synd_ref, values_ref,
                   out_ref, best_val, best_idx, sem):
    i = pl.program_id(0)
    sims = jnp.dot(keys_ref[...], anchor_ref[...],
                   preferred_element_type=jnp.float32,
                   precision=lax.Precision.HIGHEST)
    sims = jnp.where(synd_ref[...] != 0.0, jnp.float32(-1e9), sims)
    m = jnp.max(sims)
    row = lax.broadcasted_iota(jnp.int32, sims.shape, 0)
    loc = jnp.min(jnp.where(sims == m, row, jnp.int32(2**30)))

    @pl.when((i == 0) | (m > best_val[0]))
    def _():
        best_val[0] = m
        best_idx[0] = loc + i * BR

    @pl.when(i == NBLK - 1)
    def _():
        idx = best_idx[0]
        cp = pltpu.make_async_copy(values_ref.at[idx], out_ref.at[0], sem)
        cp.start()
        cp.wait()


def kernel(query, memory_keys, memory_values, syndromes):
    q64 = query.reshape(64, 8)
    anchor = pl.pallas_call(
        _anchor_body,
        out_shape=jax.ShapeDtypeStruct((64, 8), jnp.float32),
    )(q64)
    anchor_col = anchor.reshape(DIM, 1)
    synd_col = syndromes.astype(jnp.float32).reshape(CAP, 1)

    out = pl.pallas_call(
        _retrieve_body,
        grid=(NBLK,),
        in_specs=[
            pl.BlockSpec((DIM, 1), lambda i: (0, 0)),
            pl.BlockSpec((BR, DIM), lambda i: (i, 0)),
            pl.BlockSpec((BR, 1), lambda i: (i, 0)),
            pl.BlockSpec(memory_space=pl.ANY),
        ],
        out_specs=pl.BlockSpec((1, DIM), lambda i: (0, 0)),
        out_shape=jax.ShapeDtypeStruct((1, DIM), jnp.float32),
        scratch_shapes=[
            pltpu.SMEM((1,), jnp.float32),
            pltpu.SMEM((1,), jnp.int32),
            pltpu.SemaphoreType.DMA,
        ],
    )(anchor_col, memory_keys, synd_col, memory_values)
    return out.reshape(DIM)


# speedup vs baseline: 1.7433x; 1.7433x over previous
import jax
import jax.numpy as jnp
from jax import lax
from jax.experimental import pallas as pl
from jax.experimental.pallas import tpu as pltpu

DIM = 512
CAP = 100000
BR = 2000
NBLK = CAP // BR


def _nearest_d8_2d(x):
    f = jnp.round(x)
    d = x - f
    ad = jnp.abs(d)
    m = jnp.max(ad, axis=-1, keepdims=True)
    col = lax.broadcasted_iota(jnp.int32, x.shape, 1)
    k = jnp.min(jnp.where(ad == m, col, jnp.int32(8)), axis=-1, keepdims=True)
    oh = (col == k).astype(x.dtype)
    d_k = jnp.sum(d * oh, axis=-1, keepdims=True)
    corr = jnp.where(d_k >= 0.0, 1.0, -1.0)
    f_fixed = f + oh * corr
    parity_odd = jnp.mod(jnp.sum(f, axis=-1, keepdims=True), 2.0) != 0.0
    return jnp.where(parity_odd, f_fixed, f)


def _anchor_body(q_ref, out_ref):
    x = q_ref[...] * 10.0
    y0 = _nearest_d8_2d(x)
    y1 = _nearest_d8_2d(x - 0.5) + 0.5
    d0 = jnp.sum((x - y0) ** 2, axis=-1, keepdims=True)
    d1 = jnp.sum((x - y1) ** 2, axis=-1, keepdims=True)
    out_ref[...] = jnp.where(d0 <= d1, y0, y1) * 0.1


def _retrieve_body(anchor_ref, keys_ref, synd_ref, values_ref,
                   out_ref, best_val, best_idx, sem):
    i = pl.program_id(0)
    sims = jnp.dot(keys_ref[...], anchor_ref[...],
                   preferred_element_type=jnp.float32)
    sims = jnp.where(synd_ref[...] != 0.0, jnp.float32(-1e9), sims)
    m = jnp.max(sims)
    row = lax.broadcasted_iota(jnp.int32, sims.shape, 0)
    loc = jnp.min(jnp.where(sims == m, row, jnp.int32(2**30)))

    @pl.when((i == 0) | (m > best_val[0]))
    def _():
        best_val[0] = m
        best_idx[0] = loc + i * BR

    @pl.when(i == NBLK - 1)
    def _():
        idx = best_idx[0]
        cp = pltpu.make_async_copy(values_ref.at[idx], out_ref.at[0], sem)
        cp.start()
        cp.wait()


def kernel(query, memory_keys, memory_values, syndromes):
    q64 = query.reshape(64, 8)
    anchor = pl.pallas_call(
        _anchor_body,
        out_shape=jax.ShapeDtypeStruct((64, 8), jnp.float32),
    )(q64)
    anchor_col = anchor.reshape(DIM, 1)
    synd_col = syndromes.astype(jnp.float32).reshape(CAP, 1)

    out = pl.pallas_call(
        _retrieve_body,
        grid=(NBLK,),
        in_specs=[
            pl.BlockSpec((DIM, 1), lambda i: (0, 0)),
            pl.BlockSpec((BR, DIM), lambda i: (i, 0)),
            pl.BlockSpec((BR, 1), lambda i: (i, 0)),
            pl.BlockSpec(memory_space=pl.ANY),
        ],
        out_specs=pl.BlockSpec((1, DIM), lambda i: (0, 0)),
        out_shape=jax.ShapeDtypeStruct((1, DIM), jnp.float32),
        scratch_shapes=[
            pltpu.SMEM((1,), jnp.float32),
            pltpu.SMEM((1,), jnp.int32),
            pltpu.SemaphoreType.DMA,
        ],
    )(anchor_col, memory_keys, synd_col, memory_values)
    return out.reshape(DIM)


# speedup vs baseline: 3.5798x; 2.0535x over previous
import jax
import jax.numpy as jnp
from jax import lax
from jax.experimental import pallas as pl
from jax.experimental.pallas import tpu as pltpu

DIM = 512
CAP = 100000
BR = 5000
NBLK = CAP // BR


def _nearest_d8_2d(x):
    f = jnp.round(x)
    d = x - f
    ad = jnp.abs(d)
    m = jnp.max(ad, axis=-1, keepdims=True)
    col = lax.broadcasted_iota(jnp.int32, x.shape, 1)
    k = jnp.min(jnp.where(ad == m, col, jnp.int32(8)), axis=-1, keepdims=True)
    oh = (col == k).astype(x.dtype)
    d_k = jnp.sum(d * oh, axis=-1, keepdims=True)
    corr = jnp.where(d_k >= 0.0, 1.0, -1.0)
    f_fixed = f + oh * corr
    parity_odd = jnp.mod(jnp.sum(f, axis=-1, keepdims=True), 2.0) != 0.0
    return jnp.where(parity_odd, f_fixed, f)


def _anchor_body(q_ref, out_ref):
    x = q_ref[...] * 10.0
    y0 = _nearest_d8_2d(x)
    y1 = _nearest_d8_2d(x - 0.5) + 0.5
    d0 = jnp.sum((x - y0) ** 2, axis=-1, keepdims=True)
    d1 = jnp.sum((x - y1) ** 2, axis=-1, keepdims=True)
    out_ref[...] = jnp.where(d0 <= d1, y0, y1) * 0.1


def _retrieve_body(anchor_ref, keys_ref, synd_ref, values_ref,
                   out_ref, best_val, best_idx, sem):
    i = pl.program_id(0)
    sims = lax.dot_general(anchor_ref[...], keys_ref[...],
                           (((1,), (1,)), ((), ())),
                           preferred_element_type=jnp.float32)
    sims = jnp.where(synd_ref[0] != 0.0, jnp.float32(-1e9), sims)
    m = jnp.max(sims)
    col = lax.broadcasted_iota(jnp.int32, sims.shape, 1)
    loc = jnp.min(jnp.where(sims == m, col, jnp.int32(2**30)))

    @pl.when((i == 0) | (m > best_val[0]))
    def _():
        best_val[0] = m
        best_idx[0] = loc + i * BR

    @pl.when(i == NBLK - 1)
    def _():
        idx = best_idx[0]
        cp = pltpu.make_async_copy(values_ref.at[idx], out_ref.at[0], sem)
        cp.start()
        cp.wait()


def kernel(query, memory_keys, memory_values, syndromes):
    q64 = query.reshape(64, 8)
    anchor = pl.pallas_call(
        _anchor_body,
        out_shape=jax.ShapeDtypeStruct((64, 8), jnp.float32),
    )(q64)
    anchor_row = anchor.reshape(1, DIM)
    synd3 = syndromes.astype(jnp.float32).reshape(NBLK, 1, BR)

    out = pl.pallas_call(
        _retrieve_body,
        grid=(NBLK,),
        in_specs=[
            pl.BlockSpec((1, DIM), lambda i: (0, 0)),
            pl.BlockSpec((BR, DIM), lambda i: (i, 0)),
            pl.BlockSpec((1, 1, BR), lambda i: (i, 0, 0)),
            pl.BlockSpec(memory_space=pl.ANY),
        ],
        out_specs=pl.BlockSpec((1, DIM), lambda i: (0, 0)),
        out_shape=jax.ShapeDtypeStruct((1, DIM), jnp.float32),
        scratch_shapes=[
            pltpu.SMEM((1,), jnp.float32),
            pltpu.SMEM((1,), jnp.int32),
            pltpu.SemaphoreType.DMA,
        ],
    )(anchor_row, memory_keys, synd3, memory_values)
    return out.reshape(DIM)
